# baseline (speedup 1.0000x reference)
import functools

import jax
import jax.numpy as jnp
from jax import lax
from jax.experimental import pallas as pl
from jax.experimental.pallas import tpu as pltpu
from jax.experimental.pallas import tpu_sc as plsc

DIM = 768
HIDDEN = 2048
NUM_EXPERTS = 8
TOP_K = 2
N_TOK = 2048

TM = 128
M_ASN = N_TOK * TOP_K
N_TILES = (M_ASN + NUM_EXPERTS * (TM - 1) + TM - 1) // TM
M_CAP = N_TILES * TM

T_PAD = ((N_TILES + 15) // 16) * 16

NW = 32
G_ROWS = M_CAP // NW
G_CHUNK = 96
C_ROWS = N_TOK // NW

_SC_MESH = dict(core_axis_name="c", subcore_axis_name="s")


def _gate_body(x_ref, wg_ref, i1_ref, i2_ref, w1_ref, w2_ref):
    scores = lax.dot_general(x_ref[...], wg_ref[...], (((1,), (1,)), ((), ())),
                             preferred_element_type=jnp.float32)
    iota8 = lax.broadcasted_iota(jnp.int32, (N_TOK, NUM_EXPERTS), 1)
    m1 = jnp.max(scores, axis=-1, keepdims=True)
    i1 = jnp.min(jnp.where(scores == m1, iota8, NUM_EXPERTS),
                 axis=-1, keepdims=True)
    scores2 = jnp.where(iota8 == i1, -jnp.inf, scores)
    m2 = jnp.max(scores2, axis=-1, keepdims=True)
    i2 = jnp.min(jnp.where(scores2 == m2, iota8, NUM_EXPERTS),
                 axis=-1, keepdims=True)
    e2 = jnp.exp(m2 - m1)
    wa = 1.0 / (1.0 + e2)
    i1_ref[...] = i1
    i2_ref[...] = i2
    w1_ref[...] = wa
    w2_ref[...] = 1.0 - wa


def _gate(x2d, Wg):
    return pl.pallas_call(
        _gate_body,
        out_shape=(
            jax.ShapeDtypeStruct((N_TOK, 1), jnp.int32),
            jax.ShapeDtypeStruct((N_TOK, 1), jnp.int32),
            jax.ShapeDtypeStruct((N_TOK, 1), jnp.float32),
            jax.ShapeDtypeStruct((N_TOK, 1), jnp.float32),
        ),
    )(x2d, Wg)


def _route_body(i1_hbm, i2_hbm, w1_hbm, w2_hbm,
                roww_hbm, pos_hbm, tgrp_hbm, tact_hbm, nt_hbm,
                e_v, wv, rank_v, cnt_v, off_v,
                roww_v, pos_v, tg_v, ta_v, nt_v):
    @pl.when((lax.axis_index("c") == 0) & (lax.axis_index("s") == 0))
    def _():
        pltpu.sync_copy(i1_hbm, e_v.at[pl.ds(0, N_TOK)])
        pltpu.sync_copy(i2_hbm, e_v.at[pl.ds(N_TOK, N_TOK)])
        pltpu.sync_copy(w1_hbm, wv.at[pl.ds(0, N_TOK)])
        pltpu.sync_copy(w2_hbm, wv.at[pl.ds(N_TOK, N_TOK)])
        lane = lax.iota(jnp.int32, 16)
        cnt_v[...] = jnp.zeros((16,), jnp.int32)

        def body1(i, _):
            ev = e_v[pl.ds(i * 16, 16)]
            rank = plsc.load_gather(cnt_v, [ev])
            newcnt = cnt_v[...]
            for e in range(NUM_EXPERTS):
                ms32 = (ev == e).astype(jnp.int32)
                cs = lax.cumsum(ms32, axis=0)
                rank = rank + jnp.where(ev == e, cs - 1, 0)
                newcnt = newcnt + jnp.where(lane == e, jnp.sum(ms32), 0)
            cnt_v[...] = newcnt
            rank_v[pl.ds(i * 16, 16)] = rank
            return 0

        lax.fori_loop(0, M_ASN // 16, body1, 0)

        cnt = cnt_v[...]
        rc = jnp.bitwise_and(cnt + (TM - 1), jnp.int32(-TM))
        ends = lax.cumsum(rc, axis=0)
        off = ends - rc
        off_v[...] = off
        nt_v[...] = lax.div(rc, jnp.int32(TM))
        total = jnp.sum(rc)
        for half in range(T_PAD // 16):
            tl = lane + half * 16
            post = jnp.minimum(tl * TM, total - TM)
            grp = jnp.zeros((16,), jnp.int32)
            for e in range(NUM_EXPERTS):
                end_e = jnp.sum(jnp.where(lane == e, ends, 0))
                grp = grp + (post >= end_e).astype(jnp.int32)
            tg_v[pl.ds(half * 16, 16)] = grp
            ta_v[pl.ds(half * 16, 16)] = (tl * TM < total).astype(jnp.int32)

        def bz(i, _):
            roww_v[pl.ds(i * 16, 16)] = jnp.zeros((16,), jnp.float32)
            return 0

        lax.fori_loop(0, M_CAP // 16, bz, 0)

        def body3(i, _):
            ev = e_v[pl.ds(i * 16, 16)]
            p = plsc.load_gather(off_v, [ev]) + rank_v[pl.ds(i * 16, 16)]
            pos_v[pl.ds(i * 16, 16)] = p
            plsc.store_scatter(roww_v, [p], wv[pl.ds(i * 16, 16)])
            return 0

        lax.fori_loop(0, M_ASN // 16, body3, 0)

        pltpu.sync_copy(roww_v, roww_hbm)
        pltpu.sync_copy(pos_v, pos_hbm)
        pltpu.sync_copy(tg_v, tgrp_hbm)
        pltpu.sync_copy(ta_v, tact_hbm)
        pltpu.sync_copy(nt_v, nt_hbm)


def _route(i1, i2, w1, w2):
    return pl.kernel(
        _route_body,
        out_type=(
            jax.ShapeDtypeStruct((M_CAP,), jnp.float32),
            jax.ShapeDtypeStruct((M_ASN,), jnp.int32),
            jax.ShapeDtypeStruct((T_PAD,), jnp.int32),
            jax.ShapeDtypeStruct((T_PAD,), jnp.int32),
            jax.ShapeDtypeStruct((16,), jnp.int32),
        ),
        mesh=plsc.VectorSubcoreMesh(**_SC_MESH),
        compiler_params=pltpu.CompilerParams(needs_layout_passes=False),
        scratch_types=[
            pltpu.VMEM((M_ASN,), jnp.int32),
            pltpu.VMEM((M_ASN,), jnp.float32),
            pltpu.VMEM((M_ASN,), jnp.int32),
            pltpu.VMEM((16,), jnp.int32),
            pltpu.VMEM((16,), jnp.int32),
            pltpu.VMEM((M_CAP,), jnp.float32),
            pltpu.VMEM((M_ASN,), jnp.int32),
            pltpu.VMEM((T_PAD,), jnp.int32),
            pltpu.VMEM((T_PAD,), jnp.int32),
            pltpu.VMEM((16,), jnp.int32),
        ],
    )(i1, i2, w1, w2)


def _scatter_body(x_hbm, pa_hbm, pb_hbm, xs_hbm, ia_v, ib_v, buf_v, sem):
    wid = lax.axis_index("s") * 2 + lax.axis_index("c")
    pltpu.sync_copy(pa_hbm.at[wid], ia_v)
    pltpu.sync_copy(pb_hbm.at[wid], ib_v)
    pltpu.sync_copy(x_hbm.at[pl.ds(wid * C_ROWS, C_ROWS)], buf_v)
    ca = pltpu.async_copy(buf_v, xs_hbm.at[ia_v], sem)
    cb = pltpu.async_copy(buf_v, xs_hbm.at[ib_v], sem)
    ca.wait()
    cb.wait()


def _scatter(x2d, pa, pb):
    return pl.kernel(
        _scatter_body,
        out_type=jax.ShapeDtypeStruct((M_CAP, DIM), jnp.float32),
        mesh=plsc.VectorSubcoreMesh(**_SC_MESH),
        scratch_types=[
            pltpu.VMEM((C_ROWS,), jnp.int32),
            pltpu.VMEM((C_ROWS,), jnp.int32),
            pltpu.VMEM((C_ROWS, DIM), jnp.float32),
            pltpu.SemaphoreType.DMA,
        ],
    )(x2d, pa, pb)


def _gmm_body(tgrp_ref, tact_ref, nt_ref,
              xs_ref, rw_ref, w1_hbm, w3_hbm, w2_hbm, ys_ref,
              wb1, wb3, wb2, sems):
    t = pl.program_id(0)
    cur = tgrp_ref[t]
    slot = lax.rem(cur, 2)
    active = tact_ref[t] == 1
    prev = tgrp_ref[jnp.maximum(t - 1, 0)]
    is_first = (t == 0) | (prev != cur)

    def start_dma(e, s):
        pltpu.make_async_copy(w1_hbm.at[e], wb1.at[s], sems.at[s, 0]).start()
        pltpu.make_async_copy(w3_hbm.at[e], wb3.at[s], sems.at[s, 1]).start()
        pltpu.make_async_copy(w2_hbm.at[e], wb2.at[s], sems.at[s, 2]).start()

    def wait_dma(e, s):
        pltpu.make_async_copy(w1_hbm.at[e], wb1.at[s], sems.at[s, 0]).wait()
        pltpu.make_async_copy(w3_hbm.at[e], wb3.at[s], sems.at[s, 1]).wait()
        pltpu.make_async_copy(w2_hbm.at[e], wb2.at[s], sems.at[s, 2]).wait()

    @pl.when((t == 0) & active)
    def _():
        start_dma(cur, slot)

    @pl.when(is_first & active)
    def _():
        nxt = jnp.int32(9)
        for e in range(NUM_EXPERTS - 1, 0, -1):
            nxt = jnp.where((e > cur) & (nt_ref[e] > 0), jnp.int32(e), nxt)

        @pl.when(nxt < 9)
        def _():
            start_dma(nxt, 1 - slot)

    @pl.when(is_first & active)
    def _():
        wait_dma(cur, slot)

    @pl.when(active)
    def _():
        xb = xs_ref[...]
        w1b = wb1[slot]
        w3b = wb3[slot]
        w2b = wb2[slot]
        p1 = lax.dot_general(xb, w1b, (((1,), (1,)), ((), ())),
                             preferred_element_type=jnp.float32)
        p3 = lax.dot_general(xb, w3b, (((1,), (1,)), ((), ())),
                             preferred_element_type=jnp.float32)
        hh = (p1 / (1.0 + jnp.exp(-p1))) * p3
        y = lax.dot_general(hh, w2b, (((1,), (1,)), ((), ())),
                            preferred_element_type=jnp.float32)
        ys_ref[...] = y * rw_ref[...]


def _gmm(xs, roww, W1, W2, W3, tgrp, tact, nt):
    rw = roww.reshape(M_CAP, 1)
    grid_spec = pltpu.PrefetchScalarGridSpec(
        num_scalar_prefetch=3,
        grid=(N_TILES,),
        in_specs=[
            pl.BlockSpec((TM, DIM), lambda t, tg, ta, nt: (t, 0)),
            pl.BlockSpec((TM, 1), lambda t, tg, ta, nt: (t, 0)),
            pl.BlockSpec(memory_space=pl.ANY),
            pl.BlockSpec(memory_space=pl.ANY),
            pl.BlockSpec(memory_space=pl.ANY),
        ],
        out_specs=pl.BlockSpec((TM, DIM), lambda t, tg, ta, nt: (t, 0)),
        scratch_shapes=[
            pltpu.VMEM((2, HIDDEN, DIM), jnp.float32),
            pltpu.VMEM((2, HIDDEN, DIM), jnp.float32),
            pltpu.VMEM((2, DIM, HIDDEN), jnp.float32),
            pltpu.SemaphoreType.DMA((2, 3)),
        ],
    )
    return pl.pallas_call(
        _gmm_body,
        grid_spec=grid_spec,
        out_shape=jax.ShapeDtypeStruct((M_CAP, DIM), jnp.float32),
    )(tgrp, tact, nt, xs, rw, W1, W3, W2)


def _combine_body(ys_hbm, pa_hbm, pb_hbm, out_hbm, ia_v, ib_v, ba_v, bb_v,
                  sem):
    wid = lax.axis_index("s") * 2 + lax.axis_index("c")
    pltpu.sync_copy(pa_hbm.at[wid], ia_v)
    pltpu.sync_copy(pb_hbm.at[wid], ib_v)
    pltpu.async_copy(ys_hbm.at[ia_v], ba_v, sem).wait()
    pltpu.async_copy(ys_hbm.at[ib_v], bb_v, sem).wait()

    def rowbody(r, _):
        for ci in range(DIM // 16):
            sl = pl.ds(ci * 16, 16)
            ba_v[r, sl] = ba_v[r, sl] + bb_v[r, sl]
        return 0

    lax.fori_loop(0, C_ROWS, rowbody, 0)
    pltpu.sync_copy(ba_v, out_hbm.at[pl.ds(wid * C_ROWS, C_ROWS)])


def _combine(ys, pa, pb):
    return pl.kernel(
        _combine_body,
        out_type=jax.ShapeDtypeStruct((N_TOK, DIM), jnp.float32),
        mesh=plsc.VectorSubcoreMesh(**_SC_MESH),
        scratch_types=[
            pltpu.VMEM((C_ROWS,), jnp.int32),
            pltpu.VMEM((C_ROWS,), jnp.int32),
            pltpu.VMEM((C_ROWS, DIM), jnp.float32),
            pltpu.VMEM((C_ROWS, DIM), jnp.float32),
            pltpu.SemaphoreType.DMA,
        ],
    )(ys, pa, pb)


@jax.jit
def kernel(x, Wg, W1, W2, W3):
    b, s, d = x.shape
    x2d = x.reshape(b * s, d)
    i1, i2, w1, w2 = _gate(x2d, Wg)
    roww, pos, tgrp, tact, nt = _route(
        i1.reshape(-1), i2.reshape(-1), w1.reshape(-1), w2.reshape(-1))
    pa = pos[:N_TOK].reshape(NW, C_ROWS)
    pb = pos[N_TOK:].reshape(NW, C_ROWS)
    xs = _scatter(x2d, pa, pb)
    ys = _gmm(xs, roww, W1, W2, W3, tgrp, tact, nt)
    out = _combine(ys, pa, pb)
    return out.reshape(b, s, d)


# speedup vs baseline: 1.2367x; 1.2367x over previous
import functools

import jax
import jax.numpy as jnp
from jax import lax
from jax.experimental import pallas as pl
from jax.experimental.pallas import tpu as pltpu
from jax.experimental.pallas import tpu_sc as plsc

DIM = 768
HIDDEN = 2048
NUM_EXPERTS = 8
TOP_K = 2
N_TOK = 2048

TM = 256
M_ASN = N_TOK * TOP_K
N_TILES = (M_ASN + NUM_EXPERTS * (TM - 1) + TM - 1) // TM
M_CAP = N_TILES * TM

T_PAD = ((N_TILES + 15) // 16) * 16

NW = 32
G_ROWS = M_CAP // NW
G_CHUNK = 96
C_ROWS = N_TOK // NW

_SC_MESH = dict(core_axis_name="c", subcore_axis_name="s")


def _gate_body(x_ref, wg_ref, i1_ref, i2_ref, w1_ref, w2_ref):
    scores = lax.dot_general(x_ref[...], wg_ref[...], (((1,), (1,)), ((), ())),
                             preferred_element_type=jnp.float32)
    iota8 = lax.broadcasted_iota(jnp.int32, (N_TOK, NUM_EXPERTS), 1)
    m1 = jnp.max(scores, axis=-1, keepdims=True)
    i1 = jnp.min(jnp.where(scores == m1, iota8, NUM_EXPERTS),
                 axis=-1, keepdims=True)
    scores2 = jnp.where(iota8 == i1, -jnp.inf, scores)
    m2 = jnp.max(scores2, axis=-1, keepdims=True)
    i2 = jnp.min(jnp.where(scores2 == m2, iota8, NUM_EXPERTS),
                 axis=-1, keepdims=True)
    e2 = jnp.exp(m2 - m1)
    wa = 1.0 / (1.0 + e2)
    i1_ref[...] = i1
    i2_ref[...] = i2
    w1_ref[...] = wa
    w2_ref[...] = 1.0 - wa


def _gate(x2d, Wg):
    return pl.pallas_call(
        _gate_body,
        out_shape=(
            jax.ShapeDtypeStruct((N_TOK, 1), jnp.int32),
            jax.ShapeDtypeStruct((N_TOK, 1), jnp.int32),
            jax.ShapeDtypeStruct((N_TOK, 1), jnp.float32),
            jax.ShapeDtypeStruct((N_TOK, 1), jnp.float32),
        ),
    )(x2d, Wg)


def _route_body(i1_hbm, i2_hbm, w1_hbm, w2_hbm,
                roww_hbm, pos_hbm, tgrp_hbm, tact_hbm, nt_hbm,
                e_v, wv, rank_v, cnt_v, off_v,
                roww_v, pos_v, tg_v, ta_v, nt_v):
    @pl.when((lax.axis_index("c") == 0) & (lax.axis_index("s") == 0))
    def _():
        pltpu.sync_copy(i1_hbm, e_v.at[pl.ds(0, N_TOK)])
        pltpu.sync_copy(i2_hbm, e_v.at[pl.ds(N_TOK, N_TOK)])
        pltpu.sync_copy(w1_hbm, wv.at[pl.ds(0, N_TOK)])
        pltpu.sync_copy(w2_hbm, wv.at[pl.ds(N_TOK, N_TOK)])
        lane = lax.iota(jnp.int32, 16)
        cnt_v[...] = jnp.zeros((16,), jnp.int32)

        def body1(i, _):
            ev = e_v[pl.ds(i * 16, 16)]
            rank = plsc.load_gather(cnt_v, [ev])
            newcnt = cnt_v[...]
            for e in range(NUM_EXPERTS):
                ms32 = (ev == e).astype(jnp.int32)
                cs = lax.cumsum(ms32, axis=0)
                rank = rank + jnp.where(ev == e, cs - 1, 0)
                newcnt = newcnt + jnp.where(lane == e, jnp.sum(ms32), 0)
            cnt_v[...] = newcnt
            rank_v[pl.ds(i * 16, 16)] = rank
            return 0

        lax.fori_loop(0, M_ASN // 16, body1, 0)

        cnt = cnt_v[...]
        rc = jnp.bitwise_and(cnt + (TM - 1), jnp.int32(-TM))
        ends = lax.cumsum(rc, axis=0)
        off = ends - rc
        off_v[...] = off
        nt_v[...] = lax.div(rc, jnp.int32(TM))
        total = jnp.sum(rc)
        for half in range(T_PAD // 16):
            tl = lane + half * 16
            post = jnp.minimum(tl * TM, total - TM)
            grp = jnp.zeros((16,), jnp.int32)
            for e in range(NUM_EXPERTS):
                end_e = jnp.sum(jnp.where(lane == e, ends, 0))
                grp = grp + (post >= end_e).astype(jnp.int32)
            tg_v[pl.ds(half * 16, 16)] = grp
            ta_v[pl.ds(half * 16, 16)] = (tl * TM < total).astype(jnp.int32)

        def bz(i, _):
            roww_v[pl.ds(i * 16, 16)] = jnp.zeros((16,), jnp.float32)
            return 0

        lax.fori_loop(0, M_CAP // 16, bz, 0)

        def body3(i, _):
            ev = e_v[pl.ds(i * 16, 16)]
            p = plsc.load_gather(off_v, [ev]) + rank_v[pl.ds(i * 16, 16)]
            pos_v[pl.ds(i * 16, 16)] = p
            plsc.store_scatter(roww_v, [p], wv[pl.ds(i * 16, 16)])
            return 0

        lax.fori_loop(0, M_ASN // 16, body3, 0)

        pltpu.sync_copy(roww_v, roww_hbm)
        pltpu.sync_copy(pos_v, pos_hbm)
        pltpu.sync_copy(tg_v, tgrp_hbm)
        pltpu.sync_copy(ta_v, tact_hbm)
        pltpu.sync_copy(nt_v, nt_hbm)


def _route(i1, i2, w1, w2):
    return pl.kernel(
        _route_body,
        out_type=(
            jax.ShapeDtypeStruct((M_CAP,), jnp.float32),
            jax.ShapeDtypeStruct((M_ASN,), jnp.int32),
            jax.ShapeDtypeStruct((T_PAD,), jnp.int32),
            jax.ShapeDtypeStruct((T_PAD,), jnp.int32),
            jax.ShapeDtypeStruct((16,), jnp.int32),
        ),
        mesh=plsc.VectorSubcoreMesh(**_SC_MESH),
        compiler_params=pltpu.CompilerParams(needs_layout_passes=False),
        scratch_types=[
            pltpu.VMEM((M_ASN,), jnp.int32),
            pltpu.VMEM((M_ASN,), jnp.float32),
            pltpu.VMEM((M_ASN,), jnp.int32),
            pltpu.VMEM((16,), jnp.int32),
            pltpu.VMEM((16,), jnp.int32),
            pltpu.VMEM((M_CAP,), jnp.float32),
            pltpu.VMEM((M_ASN,), jnp.int32),
            pltpu.VMEM((T_PAD,), jnp.int32),
            pltpu.VMEM((T_PAD,), jnp.int32),
            pltpu.VMEM((16,), jnp.int32),
        ],
    )(i1, i2, w1, w2)


def _scatter_body(x_hbm, pa_hbm, pb_hbm, xs_hbm, ia_v, ib_v, buf_v, sem):
    wid = lax.axis_index("s") * 2 + lax.axis_index("c")
    pltpu.sync_copy(pa_hbm.at[wid], ia_v)
    pltpu.sync_copy(pb_hbm.at[wid], ib_v)
    pltpu.sync_copy(x_hbm.at[pl.ds(wid * C_ROWS, C_ROWS)], buf_v)
    ca = pltpu.async_copy(buf_v, xs_hbm.at[ia_v], sem)
    cb = pltpu.async_copy(buf_v, xs_hbm.at[ib_v], sem)
    ca.wait()
    cb.wait()


def _scatter(x2d, pa, pb):
    return pl.kernel(
        _scatter_body,
        out_type=jax.ShapeDtypeStruct((M_CAP, DIM), jnp.float32),
        mesh=plsc.VectorSubcoreMesh(**_SC_MESH),
        scratch_types=[
            pltpu.VMEM((C_ROWS,), jnp.int32),
            pltpu.VMEM((C_ROWS,), jnp.int32),
            pltpu.VMEM((C_ROWS, DIM), jnp.float32),
            pltpu.SemaphoreType.DMA,
        ],
    )(x2d, pa, pb)


def _gmm_body(tgrp_ref, tact_ref, nt_ref,
              xs_ref, rw_ref, w1_hbm, w3_hbm, w2_hbm, ys_ref,
              wb1, wb3, wb2, sems):
    t = pl.program_id(0)
    cur = tgrp_ref[t]
    slot = lax.rem(cur, 2)
    active = tact_ref[t] == 1
    prev = tgrp_ref[jnp.maximum(t - 1, 0)]
    is_first = (t == 0) | (prev != cur)

    def start_dma(e, s):
        pltpu.make_async_copy(w1_hbm.at[e], wb1.at[s], sems.at[s, 0]).start()
        pltpu.make_async_copy(w3_hbm.at[e], wb3.at[s], sems.at[s, 1]).start()
        pltpu.make_async_copy(w2_hbm.at[e], wb2.at[s], sems.at[s, 2]).start()

    @pl.when((t == 0) & active)
    def _():
        start_dma(cur, slot)

    @pl.when(is_first & active)
    def _():
        nxt = jnp.int32(9)
        for e in range(NUM_EXPERTS - 1, 0, -1):
            nxt = jnp.where((e > cur) & (nt_ref[e] > 0), jnp.int32(e), nxt)

        @pl.when(nxt < 9)
        def _():
            start_dma(nxt, 1 - slot)

    @pl.when(active)
    def _():
        xb = xs_ref[...]

        @pl.when(is_first)
        def _():
            pltpu.make_async_copy(w1_hbm.at[cur], wb1.at[slot],
                                  sems.at[slot, 0]).wait()

        p1 = lax.dot_general(xb, wb1[slot], (((1,), (1,)), ((), ())),
                             preferred_element_type=jnp.float32)

        @pl.when(is_first)
        def _():
            pltpu.make_async_copy(w3_hbm.at[cur], wb3.at[slot],
                                  sems.at[slot, 1]).wait()

        p3 = lax.dot_general(xb, wb3[slot], (((1,), (1,)), ((), ())),
                             preferred_element_type=jnp.float32)
        hh = (p1 / (1.0 + jnp.exp(-p1))) * p3

        @pl.when(is_first)
        def _():
            pltpu.make_async_copy(w2_hbm.at[cur], wb2.at[slot],
                                  sems.at[slot, 2]).wait()

        y = lax.dot_general(hh, wb2[slot], (((1,), (1,)), ((), ())),
                            preferred_element_type=jnp.float32)
        ys_ref[...] = y * rw_ref[...]


def _gmm(xs, roww, W1, W2, W3, tgrp, tact, nt):
    rw = roww.reshape(M_CAP, 1)
    grid_spec = pltpu.PrefetchScalarGridSpec(
        num_scalar_prefetch=3,
        grid=(N_TILES,),
        in_specs=[
            pl.BlockSpec((TM, DIM), lambda t, tg, ta, nt: (t, 0)),
            pl.BlockSpec((TM, 1), lambda t, tg, ta, nt: (t, 0)),
            pl.BlockSpec(memory_space=pl.ANY),
            pl.BlockSpec(memory_space=pl.ANY),
            pl.BlockSpec(memory_space=pl.ANY),
        ],
        out_specs=pl.BlockSpec((TM, DIM), lambda t, tg, ta, nt: (t, 0)),
        scratch_shapes=[
            pltpu.VMEM((2, HIDDEN, DIM), jnp.float32),
            pltpu.VMEM((2, HIDDEN, DIM), jnp.float32),
            pltpu.VMEM((2, DIM, HIDDEN), jnp.float32),
            pltpu.SemaphoreType.DMA((2, 3)),
        ],
    )
    return pl.pallas_call(
        _gmm_body,
        grid_spec=grid_spec,
        out_shape=jax.ShapeDtypeStruct((M_CAP, DIM), jnp.float32),
    )(tgrp, tact, nt, xs, rw, W1, W3, W2)


def _combine_body(ys_hbm, pa_hbm, pb_hbm, out_hbm, ia_v, ib_v, ba_v, bb_v,
                  sem):
    wid = lax.axis_index("s") * 2 + lax.axis_index("c")
    pltpu.sync_copy(pa_hbm.at[wid], ia_v)
    pltpu.sync_copy(pb_hbm.at[wid], ib_v)
    pltpu.async_copy(ys_hbm.at[ia_v], ba_v, sem).wait()
    pltpu.async_copy(ys_hbm.at[ib_v], bb_v, sem).wait()

    def rowbody(r, _):
        for ci in range(DIM // 16):
            sl = pl.ds(ci * 16, 16)
            ba_v[r, sl] = ba_v[r, sl] + bb_v[r, sl]
        return 0

    lax.fori_loop(0, C_ROWS, rowbody, 0)
    pltpu.sync_copy(ba_v, out_hbm.at[pl.ds(wid * C_ROWS, C_ROWS)])


def _combine(ys, pa, pb):
    return pl.kernel(
        _combine_body,
        out_type=jax.ShapeDtypeStruct((N_TOK, DIM), jnp.float32),
        mesh=plsc.VectorSubcoreMesh(**_SC_MESH),
        scratch_types=[
            pltpu.VMEM((C_ROWS,), jnp.int32),
            pltpu.VMEM((C_ROWS,), jnp.int32),
            pltpu.VMEM((C_ROWS, DIM), jnp.float32),
            pltpu.VMEM((C_ROWS, DIM), jnp.float32),
            pltpu.SemaphoreType.DMA,
        ],
    )(ys, pa, pb)


@jax.jit
def kernel(x, Wg, W1, W2, W3):
    b, s, d = x.shape
    x2d = x.reshape(b * s, d)
    i1, i2, w1, w2 = _gate(x2d, Wg)
    roww, pos, tgrp, tact, nt = _route(
        i1.reshape(-1), i2.reshape(-1), w1.reshape(-1), w2.reshape(-1))
    pa = pos[:N_TOK].reshape(NW, C_ROWS)
    pb = pos[N_TOK:].reshape(NW, C_ROWS)
    xs = _scatter(x2d, pa, pb)
    ys = _gmm(xs, roww, W1, W2, W3, tgrp, tact, nt)
    out = _combine(ys, pa, pb)
    return out.reshape(b, s, d)


# speedup vs baseline: 1.3388x; 1.0825x over previous
import functools

import jax
import jax.numpy as jnp
from jax import lax
from jax.experimental import pallas as pl
from jax.experimental.pallas import tpu as pltpu
from jax.experimental.pallas import tpu_sc as plsc

DIM = 768
HIDDEN = 2048
NUM_EXPERTS = 8
TOP_K = 2
N_TOK = 2048

TM = 256
M_ASN = N_TOK * TOP_K
N_TILES = (M_ASN + NUM_EXPERTS * (TM - 1) + TM - 1) // TM
M_CAP = N_TILES * TM

T_PAD = ((N_TILES + 15) // 16) * 16

NW = 32
G_ROWS = M_CAP // NW
G_CHUNK = 96
C_ROWS = N_TOK // NW

_SC_MESH = dict(core_axis_name="c", subcore_axis_name="s")


def _gate_body(x_ref, wg_ref, i1_ref, i2_ref, w1_ref, w2_ref):
    scores = lax.dot_general(x_ref[...], wg_ref[...], (((1,), (1,)), ((), ())),
                             preferred_element_type=jnp.float32)
    iota8 = lax.broadcasted_iota(jnp.int32, (N_TOK, NUM_EXPERTS), 1)
    m1 = jnp.max(scores, axis=-1, keepdims=True)
    i1 = jnp.min(jnp.where(scores == m1, iota8, NUM_EXPERTS),
                 axis=-1, keepdims=True)
    scores2 = jnp.where(iota8 == i1, -jnp.inf, scores)
    m2 = jnp.max(scores2, axis=-1, keepdims=True)
    i2 = jnp.min(jnp.where(scores2 == m2, iota8, NUM_EXPERTS),
                 axis=-1, keepdims=True)
    e2 = jnp.exp(m2 - m1)
    wa = 1.0 / (1.0 + e2)
    i1_ref[...] = i1
    i2_ref[...] = i2
    w1_ref[...] = wa
    w2_ref[...] = 1.0 - wa


def _gate(x2d, Wg):
    return pl.pallas_call(
        _gate_body,
        out_shape=(
            jax.ShapeDtypeStruct((N_TOK, 1), jnp.int32),
            jax.ShapeDtypeStruct((N_TOK, 1), jnp.int32),
            jax.ShapeDtypeStruct((N_TOK, 1), jnp.float32),
            jax.ShapeDtypeStruct((N_TOK, 1), jnp.float32),
        ),
    )(x2d, Wg)


def _route_body(i1_hbm, i2_hbm, w1_hbm, w2_hbm,
                roww_hbm, pos_hbm, tgrp_hbm, tact_hbm, nt_hbm,
                e_v, wv, rank_v, cnt_v, off_v,
                roww_v, pos_v, tg_v, ta_v, nt_v):
    @pl.when((lax.axis_index("c") == 0) & (lax.axis_index("s") == 0))
    def _():
        pltpu.sync_copy(i1_hbm, e_v.at[pl.ds(0, N_TOK)])
        pltpu.sync_copy(i2_hbm, e_v.at[pl.ds(N_TOK, N_TOK)])
        pltpu.sync_copy(w1_hbm, wv.at[pl.ds(0, N_TOK)])
        pltpu.sync_copy(w2_hbm, wv.at[pl.ds(N_TOK, N_TOK)])
        lane = lax.iota(jnp.int32, 16)
        cnt_v[...] = jnp.zeros((16,), jnp.int32)

        def body1(i, _):
            ev = e_v[pl.ds(i * 16, 16)]
            rank = plsc.load_gather(cnt_v, [ev])
            newcnt = cnt_v[...]
            for e in range(NUM_EXPERTS):
                ms32 = (ev == e).astype(jnp.int32)
                cs = lax.cumsum(ms32, axis=0)
                rank = rank + jnp.where(ev == e, cs - 1, 0)
                newcnt = newcnt + jnp.where(lane == e, jnp.sum(ms32), 0)
            cnt_v[...] = newcnt
            rank_v[pl.ds(i * 16, 16)] = rank
            return 0

        lax.fori_loop(0, M_ASN // 16, body1, 0)

        cnt = cnt_v[...]
        rc = jnp.bitwise_and(cnt + (TM - 1), jnp.int32(-TM))
        ends = lax.cumsum(rc, axis=0)
        off = ends - rc
        off_v[...] = off
        nt_v[...] = lax.div(rc, jnp.int32(TM))
        total = jnp.sum(rc)
        for half in range(T_PAD // 16):
            tl = lane + half * 16
            post = jnp.minimum(tl * TM, total - TM)
            grp = jnp.zeros((16,), jnp.int32)
            for e in range(NUM_EXPERTS):
                end_e = jnp.sum(jnp.where(lane == e, ends, 0))
                grp = grp + (post >= end_e).astype(jnp.int32)
            tg_v[pl.ds(half * 16, 16)] = grp
            ta_v[pl.ds(half * 16, 16)] = (tl * TM < total).astype(jnp.int32)

        def bz(i, _):
            roww_v[pl.ds(i * 16, 16)] = jnp.zeros((16,), jnp.float32)
            return 0

        lax.fori_loop(0, M_CAP // 16, bz, 0)

        def body3(i, _):
            ev = e_v[pl.ds(i * 16, 16)]
            p = plsc.load_gather(off_v, [ev]) + rank_v[pl.ds(i * 16, 16)]
            pos_v[pl.ds(i * 16, 16)] = p
            plsc.store_scatter(roww_v, [p], wv[pl.ds(i * 16, 16)])
            return 0

        lax.fori_loop(0, M_ASN // 16, body3, 0)

        pltpu.sync_copy(roww_v, roww_hbm)
        pltpu.sync_copy(pos_v, pos_hbm)
        pltpu.sync_copy(tg_v, tgrp_hbm)
        pltpu.sync_copy(ta_v, tact_hbm)
        pltpu.sync_copy(nt_v, nt_hbm)


def _route(i1, i2, w1, w2):
    return pl.kernel(
        _route_body,
        out_type=(
            jax.ShapeDtypeStruct((M_CAP,), jnp.float32),
            jax.ShapeDtypeStruct((M_ASN,), jnp.int32),
            jax.ShapeDtypeStruct((T_PAD,), jnp.int32),
            jax.ShapeDtypeStruct((T_PAD,), jnp.int32),
            jax.ShapeDtypeStruct((16,), jnp.int32),
        ),
        mesh=plsc.VectorSubcoreMesh(**_SC_MESH),
        compiler_params=pltpu.CompilerParams(needs_layout_passes=False),
        scratch_types=[
            pltpu.VMEM((M_ASN,), jnp.int32),
            pltpu.VMEM((M_ASN,), jnp.float32),
            pltpu.VMEM((M_ASN,), jnp.int32),
            pltpu.VMEM((16,), jnp.int32),
            pltpu.VMEM((16,), jnp.int32),
            pltpu.VMEM((M_CAP,), jnp.float32),
            pltpu.VMEM((M_ASN,), jnp.int32),
            pltpu.VMEM((T_PAD,), jnp.int32),
            pltpu.VMEM((T_PAD,), jnp.int32),
            pltpu.VMEM((16,), jnp.int32),
        ],
    )(i1, i2, w1, w2)


def _scatter_body(x_hbm, pa_hbm, pb_hbm, xs_hbm, ia_v, ib_v, buf_v, sem):
    wid = lax.axis_index("s") * 2 + lax.axis_index("c")
    pltpu.sync_copy(pa_hbm.at[wid], ia_v)
    pltpu.sync_copy(pb_hbm.at[wid], ib_v)
    pltpu.sync_copy(x_hbm.at[pl.ds(wid * C_ROWS, C_ROWS)], buf_v)
    ca = pltpu.async_copy(buf_v, xs_hbm.at[ia_v], sem)
    cb = pltpu.async_copy(buf_v, xs_hbm.at[ib_v], sem)
    ca.wait()
    cb.wait()


def _scatter(x2d, pa, pb):
    return pl.kernel(
        _scatter_body,
        out_type=jax.ShapeDtypeStruct((M_CAP, DIM), jnp.float32),
        mesh=plsc.VectorSubcoreMesh(**_SC_MESH),
        scratch_types=[
            pltpu.VMEM((C_ROWS,), jnp.int32),
            pltpu.VMEM((C_ROWS,), jnp.int32),
            pltpu.VMEM((C_ROWS, DIM), jnp.float32),
            pltpu.SemaphoreType.DMA,
        ],
    )(x2d, pa, pb)


def _gmm_body(tgrp_ref, tact_ref, nt_ref,
              xs_ref, rw_ref, w1_hbm, w3_hbm, w2_hbm, ys_ref,
              wb1, wb3, wb2, sems):
    t = pl.program_id(0)
    cur = tgrp_ref[t]
    slot = lax.rem(cur, 2)
    active = tact_ref[t] == 1
    prev = tgrp_ref[jnp.maximum(t - 1, 0)]
    is_first = (t == 0) | (prev != cur)

    def start_dma(e, s):
        pltpu.make_async_copy(w1_hbm.at[e], wb1.at[s], sems.at[s, 0]).start()
        pltpu.make_async_copy(w3_hbm.at[e], wb3.at[s], sems.at[s, 1]).start()
        pltpu.make_async_copy(w2_hbm.at[e], wb2.at[s], sems.at[s, 2]).start()

    @pl.when((t == 0) & active)
    def _():
        start_dma(cur, slot)

    @pl.when(is_first & active)
    def _():
        nxt = jnp.int32(9)
        for e in range(NUM_EXPERTS - 1, 0, -1):
            nxt = jnp.where((e > cur) & (nt_ref[e] > 0), jnp.int32(e), nxt)

        @pl.when(nxt < 9)
        def _():
            start_dma(nxt, 1 - slot)

    @pl.when(is_first & active)
    def _():
        pltpu.make_async_copy(w1_hbm.at[cur], wb1.at[slot],
                              sems.at[slot, 0]).wait()
        pltpu.make_async_copy(w3_hbm.at[cur], wb3.at[slot],
                              sems.at[slot, 1]).wait()
        pltpu.make_async_copy(w2_hbm.at[cur], wb2.at[slot],
                              sems.at[slot, 2]).wait()

    @pl.when(active)
    def _():
        xb = xs_ref[...]
        p1 = lax.dot_general(xb, wb1[slot], (((1,), (1,)), ((), ())),
                             preferred_element_type=jnp.float32)
        p3 = lax.dot_general(xb, wb3[slot], (((1,), (1,)), ((), ())),
                             preferred_element_type=jnp.float32)
        hh = (p1 / (1.0 + jnp.exp(-p1))) * p3
        y = lax.dot_general(hh, wb2[slot], (((1,), (1,)), ((), ())),
                            preferred_element_type=jnp.float32)
        ys_ref[...] = y * rw_ref[...]


def _gmm(xs, roww, W1, W2, W3, tgrp, tact, nt):
    rw = roww.reshape(M_CAP, 1)
    grid_spec = pltpu.PrefetchScalarGridSpec(
        num_scalar_prefetch=3,
        grid=(N_TILES,),
        in_specs=[
            pl.BlockSpec((TM, DIM), lambda t, tg, ta, nt: (t, 0)),
            pl.BlockSpec((TM, 1), lambda t, tg, ta, nt: (t, 0)),
            pl.BlockSpec(memory_space=pl.ANY),
            pl.BlockSpec(memory_space=pl.ANY),
            pl.BlockSpec(memory_space=pl.ANY),
        ],
        out_specs=pl.BlockSpec((TM, DIM), lambda t, tg, ta, nt: (t, 0)),
        scratch_shapes=[
            pltpu.VMEM((2, HIDDEN, DIM), jnp.float32),
            pltpu.VMEM((2, HIDDEN, DIM), jnp.float32),
            pltpu.VMEM((2, DIM, HIDDEN), jnp.float32),
            pltpu.SemaphoreType.DMA((2, 3)),
        ],
    )
    return pl.pallas_call(
        _gmm_body,
        grid_spec=grid_spec,
        out_shape=jax.ShapeDtypeStruct((M_CAP, DIM), jnp.float32),
    )(tgrp, tact, nt, xs, rw, W1, W3, W2)


def _combine_body(ys_hbm, pa_hbm, pb_hbm, out_hbm, ia_v, ib_v, ba_v, bb_v,
                  sem):
    wid = lax.axis_index("s") * 2 + lax.axis_index("c")
    pltpu.sync_copy(pa_hbm.at[wid], ia_v)
    pltpu.sync_copy(pb_hbm.at[wid], ib_v)
    pltpu.async_copy(ys_hbm.at[ia_v], ba_v, sem).wait()
    pltpu.async_copy(ys_hbm.at[ib_v], bb_v, sem).wait()

    def rowbody(r, _):
        for ci in range(DIM // 16):
            sl = pl.ds(ci * 16, 16)
            ba_v[r, sl] = ba_v[r, sl] + bb_v[r, sl]
        return 0

    lax.fori_loop(0, C_ROWS, rowbody, 0)
    pltpu.sync_copy(ba_v, out_hbm.at[pl.ds(wid * C_ROWS, C_ROWS)])


def _combine(ys, pa, pb):
    return pl.kernel(
        _combine_body,
        out_type=jax.ShapeDtypeStruct((N_TOK, DIM), jnp.float32),
        mesh=plsc.VectorSubcoreMesh(**_SC_MESH),
        scratch_types=[
            pltpu.VMEM((C_ROWS,), jnp.int32),
            pltpu.VMEM((C_ROWS,), jnp.int32),
            pltpu.VMEM((C_ROWS, DIM), jnp.float32),
            pltpu.VMEM((C_ROWS, DIM), jnp.float32),
            pltpu.SemaphoreType.DMA,
        ],
    )(ys, pa, pb)


@jax.jit
def kernel(x, Wg, W1, W2, W3):
    b, s, d = x.shape
    x2d = x.reshape(b * s, d)
    i1, i2, w1, w2 = _gate(x2d, Wg)
    roww, pos, tgrp, tact, nt = _route(
        i1.reshape(-1), i2.reshape(-1), w1.reshape(-1), w2.reshape(-1))
    pa = pos[:N_TOK].reshape(NW, C_ROWS)
    pb = pos[N_TOK:].reshape(NW, C_ROWS)
    xs = _scatter(x2d, pa, pb)
    ys = _gmm(xs, roww, W1, W2, W3, tgrp, tact, nt)
    out = _combine(ys, pa, pb)
    return out.reshape(b, s, d)


# speedup vs baseline: 1.4426x; 1.0775x over previous
import functools

import jax
import jax.numpy as jnp
from jax import lax
from jax.experimental import pallas as pl
from jax.experimental.pallas import tpu as pltpu
from jax.experimental.pallas import tpu_sc as plsc

DIM = 768
HIDDEN = 2048
NUM_EXPERTS = 8
TOP_K = 2
N_TOK = 2048

TM = 256
M_ASN = N_TOK * TOP_K
N_TILES = (M_ASN + NUM_EXPERTS * (TM - 1) + TM - 1) // TM
M_CAP = N_TILES * TM

T_PAD = ((N_TILES + 15) // 16) * 16

NW = 32
G_ROWS = M_CAP // NW
G_CHUNK = 96
C_ROWS = N_TOK // NW

_SC_MESH = dict(core_axis_name="c", subcore_axis_name="s")


def _gate_body(x_ref, wg_ref, ev_ref, wv_ref):
    scores = lax.dot_general(x_ref[...], wg_ref[...], (((1,), (1,)), ((), ())),
                             preferred_element_type=jnp.float32)
    iota8 = lax.broadcasted_iota(jnp.int32, (N_TOK, NUM_EXPERTS), 1)
    m1 = jnp.max(scores, axis=-1, keepdims=True)
    i1 = jnp.min(jnp.where(scores == m1, iota8, NUM_EXPERTS),
                 axis=-1, keepdims=True)
    scores2 = jnp.where(iota8 == i1, -jnp.inf, scores)
    m2 = jnp.max(scores2, axis=-1, keepdims=True)
    i2 = jnp.min(jnp.where(scores2 == m2, iota8, NUM_EXPERTS),
                 axis=-1, keepdims=True)
    e2 = jnp.exp(m2 - m1)
    wa = 1.0 / (1.0 + e2)
    ev_ref[pl.ds(0, N_TOK), :] = i1
    ev_ref[pl.ds(N_TOK, N_TOK), :] = i2
    wv_ref[pl.ds(0, N_TOK), :] = wa
    wv_ref[pl.ds(N_TOK, N_TOK), :] = 1.0 - wa


def _gate(x2d, Wg):
    return pl.pallas_call(
        _gate_body,
        out_shape=(
            jax.ShapeDtypeStruct((M_ASN, 1), jnp.int32),
            jax.ShapeDtypeStruct((M_ASN, 1), jnp.float32),
        ),
    )(x2d, Wg)


A_PER = M_ASN // 16


def _rs_body(ev_hbm, wv_hbm, x_hbm,
             roww_hbm, pos_hbm, tgrp_hbm, tact_hbm, nt_hbm, xs_hbm,
             e_v, rank_v, lcnt_v, all_v, off_v,
             pos_v, pos2_v, wch_v, tg_v, ta_v, nt_v, zbuf_v,
             pa_v, pb_v, xbuf_v,
             cnt_sh, pos_sh, roww_sh, sem):
    c = lax.axis_index("c")
    s = lax.axis_index("s")
    lane = lax.iota(jnp.int32, 16)

    pltpu.sync_copy(ev_hbm.at[pl.ds(s * A_PER, A_PER)], e_v)
    pltpu.sync_copy(wv_hbm.at[pl.ds(s * 2, 2)], wch_v)
    lcnt_v[...] = jnp.zeros((16,), jnp.int32)

    def body_a(i, _):
        ev = e_v[pl.ds(i * 16, 16)]
        rank = plsc.load_gather(lcnt_v, [ev])
        newcnt = lcnt_v[...]
        for e in range(NUM_EXPERTS):
            ms32 = (ev == e).astype(jnp.int32)
            cs = lax.cumsum(ms32, axis=0)
            rank = rank + jnp.where(ev == e, cs - 1, 0)
            newcnt = newcnt + jnp.where(lane == e, jnp.sum(ms32), 0)
        lcnt_v[...] = newcnt
        rank_v[pl.ds(i * 16, 16)] = rank
        return 0

    lax.fori_loop(0, A_PER // 16, body_a, 0)
    pltpu.sync_copy(lcnt_v, cnt_sh.at[s])

    @pl.when(s == 0)
    def _():
        def bz(i, _):
            zbuf_v[pl.ds(i * 16, 16)] = jnp.zeros((16,), jnp.float32)
            return 0

        lax.fori_loop(0, M_CAP // 16, bz, 0)
        pltpu.sync_copy(zbuf_v, roww_sh)

    plsc.subcore_barrier()

    pltpu.sync_copy(cnt_sh, all_v)
    cnt = jnp.zeros((16,), jnp.int32)
    base = jnp.zeros((16,), jnp.int32)
    for tt in range(16):
        row = all_v[tt]
        cnt = cnt + row
        base = base + jnp.where(jnp.int32(tt) < s, row, 0)
    rc = jnp.bitwise_and(cnt + (TM - 1), jnp.int32(-TM))
    ends = lax.cumsum(rc, axis=0)
    off = ends - rc
    off_v[...] = off + base
    total = jnp.sum(rc)

    def body_b(i, _):
        ev = e_v[pl.ds(i * 16, 16)]
        p = plsc.load_gather(off_v, [ev]) + rank_v[pl.ds(i * 16, 16)]
        pos_v[pl.ds(i * 16, 16)] = p
        pos2_v[lax.div(i, 8), pl.ds(lax.rem(i, 8) * 16, 16)] = p
        return 0

    lax.fori_loop(0, A_PER // 16, body_b, 0)
    pltpu.sync_copy(pos_v, pos_sh.at[pl.ds(s * A_PER, A_PER)])
    for half in range(2):
        pltpu.sync_copy(wch_v.at[half], roww_sh.at[pos2_v.at[half]])

    @pl.when(c == 0)
    def _():
        pltpu.sync_copy(pos_v, pos_hbm.at[pl.ds(s * A_PER, A_PER)])

    @pl.when((s == 0) & (c == 0))
    def _():
        nt_v[...] = lax.div(rc, jnp.int32(TM))
        pltpu.sync_copy(nt_v, nt_hbm)
        for half in range(T_PAD // 16):
            tl = lane + half * 16
            post = jnp.minimum(tl * TM, total - TM)
            grp = jnp.zeros((16,), jnp.int32)
            for e in range(NUM_EXPERTS):
                end_e = jnp.sum(jnp.where(lane == e, ends, 0))
                grp = grp + (post >= end_e).astype(jnp.int32)
            tg_v[pl.ds(half * 16, 16)] = grp
            ta_v[pl.ds(half * 16, 16)] = (tl * TM < total).astype(jnp.int32)
        pltpu.sync_copy(tg_v, tgrp_hbm)
        pltpu.sync_copy(ta_v, tact_hbm)

    plsc.subcore_barrier()

    @pl.when((s == 0) & (c == 0))
    def _():
        pltpu.sync_copy(roww_sh, roww_hbm)

    w = s * 2 + c
    pltpu.sync_copy(pos_sh.at[pl.ds(w * C_ROWS, C_ROWS)], pa_v)
    pltpu.sync_copy(pos_sh.at[pl.ds(N_TOK + w * C_ROWS, C_ROWS)], pb_v)
    pltpu.sync_copy(x_hbm.at[pl.ds(w * C_ROWS, C_ROWS)], xbuf_v)
    ca = pltpu.async_copy(xbuf_v, xs_hbm.at[pa_v], sem)
    cb = pltpu.async_copy(xbuf_v, xs_hbm.at[pb_v], sem)
    ca.wait()
    cb.wait()


def _route_scatter(ev, wv, x2d):
    return pl.kernel(
        _rs_body,
        out_type=(
            jax.ShapeDtypeStruct((M_CAP,), jnp.float32),
            jax.ShapeDtypeStruct((M_ASN,), jnp.int32),
            jax.ShapeDtypeStruct((T_PAD,), jnp.int32),
            jax.ShapeDtypeStruct((T_PAD,), jnp.int32),
            jax.ShapeDtypeStruct((16,), jnp.int32),
            jax.ShapeDtypeStruct((M_CAP, DIM), jnp.float32),
        ),
        mesh=plsc.VectorSubcoreMesh(**_SC_MESH),
        compiler_params=pltpu.CompilerParams(needs_layout_passes=False),
        scratch_types=[
            pltpu.VMEM((A_PER,), jnp.int32),
            pltpu.VMEM((A_PER,), jnp.int32),
            pltpu.VMEM((16,), jnp.int32),
            pltpu.VMEM((16, 16), jnp.int32),
            pltpu.VMEM((16,), jnp.int32),
            pltpu.VMEM((A_PER,), jnp.int32),
            pltpu.VMEM((2, 128), jnp.int32),
            pltpu.VMEM((2, 128), jnp.float32),
            pltpu.VMEM((T_PAD,), jnp.int32),
            pltpu.VMEM((T_PAD,), jnp.int32),
            pltpu.VMEM((16,), jnp.int32),
            pltpu.VMEM((M_CAP,), jnp.float32),
            pltpu.VMEM((C_ROWS,), jnp.int32),
            pltpu.VMEM((C_ROWS,), jnp.int32),
            pltpu.VMEM((C_ROWS, DIM), jnp.float32),
            pltpu.VMEM_SHARED((16, 16), jnp.int32),
            pltpu.VMEM_SHARED((M_ASN,), jnp.int32),
            pltpu.VMEM_SHARED((M_CAP,), jnp.float32),
            pltpu.SemaphoreType.DMA,
        ],
    )(ev, wv, x2d)


def _gmm_body(tgrp_ref, tact_ref, nt_ref,
              xs_ref, rw_ref, w1_hbm, w3_hbm, w2_hbm, ys_ref,
              wb1, wb3, wb2, sems):
    t = pl.program_id(0)
    cur = tgrp_ref[t]
    slot = lax.rem(cur, 2)
    active = tact_ref[t] == 1
    prev = tgrp_ref[jnp.maximum(t - 1, 0)]
    is_first = (t == 0) | (prev != cur)

    def start_dma(e, s):
        pltpu.make_async_copy(w1_hbm.at[e], wb1.at[s], sems.at[s, 0]).start()
        pltpu.make_async_copy(w3_hbm.at[e], wb3.at[s], sems.at[s, 1]).start()
        pltpu.make_async_copy(w2_hbm.at[e], wb2.at[s], sems.at[s, 2]).start()

    @pl.when((t == 0) & active)
    def _():
        start_dma(cur, slot)

    @pl.when(is_first & active)
    def _():
        nxt = jnp.int32(9)
        for e in range(NUM_EXPERTS - 1, 0, -1):
            nxt = jnp.where((e > cur) & (nt_ref[e] > 0), jnp.int32(e), nxt)

        @pl.when(nxt < 9)
        def _():
            start_dma(nxt, 1 - slot)

    @pl.when(is_first & active)
    def _():
        pltpu.make_async_copy(w1_hbm.at[cur], wb1.at[slot],
                              sems.at[slot, 0]).wait()
        pltpu.make_async_copy(w3_hbm.at[cur], wb3.at[slot],
                              sems.at[slot, 1]).wait()
        pltpu.make_async_copy(w2_hbm.at[cur], wb2.at[slot],
                              sems.at[slot, 2]).wait()

    @pl.when(active)
    def _():
        xb = xs_ref[...]
        p1 = lax.dot_general(xb, wb1[slot], (((1,), (1,)), ((), ())),
                             preferred_element_type=jnp.float32)
        p3 = lax.dot_general(xb, wb3[slot], (((1,), (1,)), ((), ())),
                             preferred_element_type=jnp.float32)
        hh = (p1 / (1.0 + jnp.exp(-p1))) * p3
        y = lax.dot_general(hh, wb2[slot], (((1,), (1,)), ((), ())),
                            preferred_element_type=jnp.float32)
        ys_ref[...] = y * rw_ref[...]


def _gmm(xs, roww, W1, W2, W3, tgrp, tact, nt):
    rw = roww.reshape(M_CAP, 1)
    grid_spec = pltpu.PrefetchScalarGridSpec(
        num_scalar_prefetch=3,
        grid=(N_TILES,),
        in_specs=[
            pl.BlockSpec((TM, DIM), lambda t, tg, ta, nt: (t, 0)),
            pl.BlockSpec((TM, 1), lambda t, tg, ta, nt: (t, 0)),
            pl.BlockSpec(memory_space=pl.ANY),
            pl.BlockSpec(memory_space=pl.ANY),
            pl.BlockSpec(memory_space=pl.ANY),
        ],
        out_specs=pl.BlockSpec((TM, DIM), lambda t, tg, ta, nt: (t, 0)),
        scratch_shapes=[
            pltpu.VMEM((2, HIDDEN, DIM), jnp.float32),
            pltpu.VMEM((2, HIDDEN, DIM), jnp.float32),
            pltpu.VMEM((2, DIM, HIDDEN), jnp.float32),
            pltpu.SemaphoreType.DMA((2, 3)),
        ],
    )
    return pl.pallas_call(
        _gmm_body,
        grid_spec=grid_spec,
        out_shape=jax.ShapeDtypeStruct((M_CAP, DIM), jnp.float32),
    )(tgrp, tact, nt, xs, rw, W1, W3, W2)


def _combine_body(ys_hbm, pa_hbm, pb_hbm, out_hbm, ia_v, ib_v, ba_v, bb_v,
                  sem):
    wid = lax.axis_index("s") * 2 + lax.axis_index("c")
    pltpu.sync_copy(pa_hbm.at[wid], ia_v)
    pltpu.sync_copy(pb_hbm.at[wid], ib_v)
    pltpu.async_copy(ys_hbm.at[ia_v], ba_v, sem).wait()
    pltpu.async_copy(ys_hbm.at[ib_v], bb_v, sem).wait()

    def rowbody(r, _):
        for ci in range(DIM // 16):
            sl = pl.ds(ci * 16, 16)
            ba_v[r, sl] = ba_v[r, sl] + bb_v[r, sl]
        return 0

    lax.fori_loop(0, C_ROWS, rowbody, 0)
    pltpu.sync_copy(ba_v, out_hbm.at[pl.ds(wid * C_ROWS, C_ROWS)])


def _combine(ys, pa, pb):
    return pl.kernel(
        _combine_body,
        out_type=jax.ShapeDtypeStruct((N_TOK, DIM), jnp.float32),
        mesh=plsc.VectorSubcoreMesh(**_SC_MESH),
        scratch_types=[
            pltpu.VMEM((C_ROWS,), jnp.int32),
            pltpu.VMEM((C_ROWS,), jnp.int32),
            pltpu.VMEM((C_ROWS, DIM), jnp.float32),
            pltpu.VMEM((C_ROWS, DIM), jnp.float32),
            pltpu.SemaphoreType.DMA,
        ],
    )(ys, pa, pb)


@jax.jit
def kernel(x, Wg, W1, W2, W3):
    b, s, d = x.shape
    x2d = x.reshape(b * s, d)
    ev, wv = _gate(x2d, Wg)
    roww, pos, tgrp, tact, nt, xs = _route_scatter(
        ev.reshape(-1), wv.reshape(NW, M_ASN // NW), x2d)
    pa = pos[:N_TOK].reshape(NW, C_ROWS)
    pb = pos[N_TOK:].reshape(NW, C_ROWS)
    ys = _gmm(xs, roww, W1, W2, W3, tgrp, tact, nt)
    out = _combine(ys, pa, pb)
    return out.reshape(b, s, d)


# speedup vs baseline: 1.4768x; 1.0237x over previous
import functools

import jax
import jax.numpy as jnp
from jax import lax
from jax.experimental import pallas as pl
from jax.experimental.pallas import tpu as pltpu
from jax.experimental.pallas import tpu_sc as plsc

DIM = 768
HIDDEN = 2048
NUM_EXPERTS = 8
TOP_K = 2
N_TOK = 2048

TM = 256
M_ASN = N_TOK * TOP_K
N_TILES = (M_ASN + NUM_EXPERTS * (TM - 1) + TM - 1) // TM
M_CAP = N_TILES * TM

T_PAD = ((N_TILES + 15) // 16) * 16

NW = 32
G_ROWS = M_CAP // NW
G_CHUNK = 96
C_ROWS = N_TOK // NW

_SC_MESH = dict(core_axis_name="c", subcore_axis_name="s")


def _gate_body(x_ref, wg_ref, ev_ref, wv_ref):
    scores = lax.dot_general(x_ref[...], wg_ref[...], (((1,), (1,)), ((), ())),
                             preferred_element_type=jnp.float32)
    iota8 = lax.broadcasted_iota(jnp.int32, (N_TOK, NUM_EXPERTS), 1)
    m1 = jnp.max(scores, axis=-1, keepdims=True)
    i1 = jnp.min(jnp.where(scores == m1, iota8, NUM_EXPERTS),
                 axis=-1, keepdims=True)
    scores2 = jnp.where(iota8 == i1, -jnp.inf, scores)
    m2 = jnp.max(scores2, axis=-1, keepdims=True)
    i2 = jnp.min(jnp.where(scores2 == m2, iota8, NUM_EXPERTS),
                 axis=-1, keepdims=True)
    e2 = jnp.exp(m2 - m1)
    wa = 1.0 / (1.0 + e2)
    ev_ref[pl.ds(0, N_TOK), :] = i1
    ev_ref[pl.ds(N_TOK, N_TOK), :] = i2
    wv_ref[pl.ds(0, N_TOK), :] = wa
    wv_ref[pl.ds(N_TOK, N_TOK), :] = 1.0 - wa


def _gate(x2d, Wg):
    return pl.pallas_call(
        _gate_body,
        out_shape=(
            jax.ShapeDtypeStruct((M_ASN, 1), jnp.int32),
            jax.ShapeDtypeStruct((M_ASN, 1), jnp.float32),
        ),
    )(x2d, Wg)


A_PER = M_ASN // 16


def _rs_body(ev_hbm, wv_hbm, x_hbm,
             roww_hbm, pos_hbm, tgrp_hbm, tact_hbm, nt_hbm, st_hbm, xs_hbm,
             e_v, rank_v, lcnt_v, all_v, off_v,
             pos_v, pos2_v, wch_v, tg_v, ta_v, nt_v, st_v, zbuf_v,
             pa_v, pb_v, xbuf_v,
             cnt_sh, pos_sh, roww_sh, sem):
    c = lax.axis_index("c")
    s = lax.axis_index("s")
    lane = lax.iota(jnp.int32, 16)

    pltpu.sync_copy(ev_hbm.at[pl.ds(s * A_PER, A_PER)], e_v)
    pltpu.sync_copy(wv_hbm.at[pl.ds(s * 2, 2)], wch_v)
    lcnt_v[...] = jnp.zeros((16,), jnp.int32)

    def body_a(i, _):
        ev = e_v[pl.ds(i * 16, 16)]
        rank = plsc.load_gather(lcnt_v, [ev])
        newcnt = lcnt_v[...]
        for e in range(NUM_EXPERTS):
            ms32 = (ev == e).astype(jnp.int32)
            cs = lax.cumsum(ms32, axis=0)
            rank = rank + jnp.where(ev == e, cs - 1, 0)
            newcnt = newcnt + jnp.where(lane == e, jnp.sum(ms32), 0)
        lcnt_v[...] = newcnt
        rank_v[pl.ds(i * 16, 16)] = rank
        return 0

    lax.fori_loop(0, A_PER // 16, body_a, 0)
    pltpu.sync_copy(lcnt_v, cnt_sh.at[s])

    @pl.when(s == 0)
    def _():
        def bz(i, _):
            zbuf_v[pl.ds(i * 16, 16)] = jnp.zeros((16,), jnp.float32)
            return 0

        lax.fori_loop(0, M_CAP // 16, bz, 0)
        pltpu.sync_copy(zbuf_v, roww_sh)

    plsc.subcore_barrier()

    pltpu.sync_copy(cnt_sh, all_v)
    cnt = jnp.zeros((16,), jnp.int32)
    base = jnp.zeros((16,), jnp.int32)
    for tt in range(16):
        row = all_v[tt]
        cnt = cnt + row
        base = base + jnp.where(jnp.int32(tt) < s, row, 0)
    rc = jnp.bitwise_and(cnt + (TM - 1), jnp.int32(-TM))
    ends = lax.cumsum(rc, axis=0)
    off = ends - rc
    off_v[...] = off + base
    total = jnp.sum(rc)

    def body_b(i, _):
        ev = e_v[pl.ds(i * 16, 16)]
        p = plsc.load_gather(off_v, [ev]) + rank_v[pl.ds(i * 16, 16)]
        pos_v[pl.ds(i * 16, 16)] = p
        pos2_v[lax.div(i, 8), pl.ds(lax.rem(i, 8) * 16, 16)] = p
        return 0

    lax.fori_loop(0, A_PER // 16, body_b, 0)
    pltpu.sync_copy(pos_v, pos_sh.at[pl.ds(s * A_PER, A_PER)])
    for half in range(2):
        pltpu.sync_copy(wch_v.at[half], roww_sh.at[pos2_v.at[half]])

    @pl.when(c == 0)
    def _():
        pltpu.sync_copy(pos_v, pos_hbm.at[pl.ds(s * A_PER, A_PER)])

    @pl.when((s == 0) & (c == 0))
    def _():
        nt_v[...] = lax.div(rc, jnp.int32(TM))
        pltpu.sync_copy(nt_v, nt_hbm)
        st_v[...] = lax.div(off, jnp.int32(TM))
        pltpu.sync_copy(st_v, st_hbm)
        for half in range(T_PAD // 16):
            tl = lane + half * 16
            post = jnp.minimum(tl * TM, total - TM)
            grp = jnp.zeros((16,), jnp.int32)
            for e in range(NUM_EXPERTS):
                end_e = jnp.sum(jnp.where(lane == e, ends, 0))
                grp = grp + (post >= end_e).astype(jnp.int32)
            tg_v[pl.ds(half * 16, 16)] = grp
            ta_v[pl.ds(half * 16, 16)] = (tl * TM < total).astype(jnp.int32)
        pltpu.sync_copy(tg_v, tgrp_hbm)
        pltpu.sync_copy(ta_v, tact_hbm)

    plsc.subcore_barrier()

    @pl.when((s == 0) & (c == 0))
    def _():
        pltpu.sync_copy(roww_sh, roww_hbm)

    w = s * 2 + c
    pltpu.sync_copy(pos_sh.at[pl.ds(w * C_ROWS, C_ROWS)], pa_v)
    pltpu.sync_copy(pos_sh.at[pl.ds(N_TOK + w * C_ROWS, C_ROWS)], pb_v)
    pltpu.sync_copy(x_hbm.at[pl.ds(w * C_ROWS, C_ROWS)], xbuf_v)
    ca = pltpu.async_copy(xbuf_v, xs_hbm.at[pa_v], sem)
    cb = pltpu.async_copy(xbuf_v, xs_hbm.at[pb_v], sem)
    ca.wait()
    cb.wait()


def _route_scatter(ev, wv, x2d):
    return pl.kernel(
        _rs_body,
        out_type=(
            jax.ShapeDtypeStruct((M_CAP,), jnp.float32),
            jax.ShapeDtypeStruct((M_ASN,), jnp.int32),
            jax.ShapeDtypeStruct((T_PAD,), jnp.int32),
            jax.ShapeDtypeStruct((T_PAD,), jnp.int32),
            jax.ShapeDtypeStruct((16,), jnp.int32),
            jax.ShapeDtypeStruct((16,), jnp.int32),
            jax.ShapeDtypeStruct((M_CAP, DIM), jnp.float32),
        ),
        mesh=plsc.VectorSubcoreMesh(**_SC_MESH),
        compiler_params=pltpu.CompilerParams(needs_layout_passes=False),
        scratch_types=[
            pltpu.VMEM((A_PER,), jnp.int32),
            pltpu.VMEM((A_PER,), jnp.int32),
            pltpu.VMEM((16,), jnp.int32),
            pltpu.VMEM((16, 16), jnp.int32),
            pltpu.VMEM((16,), jnp.int32),
            pltpu.VMEM((A_PER,), jnp.int32),
            pltpu.VMEM((2, 128), jnp.int32),
            pltpu.VMEM((2, 128), jnp.float32),
            pltpu.VMEM((T_PAD,), jnp.int32),
            pltpu.VMEM((T_PAD,), jnp.int32),
            pltpu.VMEM((16,), jnp.int32),
            pltpu.VMEM((16,), jnp.int32),
            pltpu.VMEM((M_CAP,), jnp.float32),
            pltpu.VMEM((C_ROWS,), jnp.int32),
            pltpu.VMEM((C_ROWS,), jnp.int32),
            pltpu.VMEM((C_ROWS, DIM), jnp.float32),
            pltpu.VMEM_SHARED((16, 16), jnp.int32),
            pltpu.VMEM_SHARED((M_ASN,), jnp.int32),
            pltpu.VMEM_SHARED((M_CAP,), jnp.float32),
            pltpu.SemaphoreType.DMA,
        ],
    )(ev, wv, x2d)


N_CHUNK = 4
TH = HIDDEN // N_CHUNK
W2_CH = DIM // N_CHUNK


def _gmm_body(tgrp_ref, tact_ref, nt_ref, st_ref,
              xs_ref, rw_ref, w1_hbm, w3_hbm, w2_hbm, ys_ref,
              wb1, wb3, wb2, sems):
    t = pl.program_id(0)
    cur = tgrp_ref[t]
    slot = lax.rem(cur, 2)
    active = tact_ref[t] == 1
    prev = tgrp_ref[jnp.maximum(t - 1, 0)]
    is_first = (t == 0) | (prev != cur)

    def start_dma(e, s):
        pltpu.make_async_copy(w1_hbm.at[e], wb1.at[s], sems.at[s, 0]).start()
        pltpu.make_async_copy(w3_hbm.at[e], wb3.at[s], sems.at[s, 1]).start()
        pltpu.make_async_copy(w2_hbm.at[e], wb2.at[s], sems.at[s, 2]).start()

    def start_chunk(e, s, i):
        arr, p = divmod(i, N_CHUNK)
        if arr == 0:
            pltpu.make_async_copy(w1_hbm.at[e, pl.ds(p * TH, TH)],
                                  wb1.at[s, pl.ds(p * TH, TH)],
                                  sems.at[s, 0]).start()
        elif arr == 1:
            pltpu.make_async_copy(w3_hbm.at[e, pl.ds(p * TH, TH)],
                                  wb3.at[s, pl.ds(p * TH, TH)],
                                  sems.at[s, 1]).start()
        else:
            pltpu.make_async_copy(w2_hbm.at[e, pl.ds(p * W2_CH, W2_CH)],
                                  wb2.at[s, pl.ds(p * W2_CH, W2_CH)],
                                  sems.at[s, 2]).start()

    @pl.when((t == 0) & active)
    def _():
        start_dma(cur, slot)

    nxt = jnp.int32(9)
    for e in range(NUM_EXPERTS - 1, 0, -1):
        nxt = jnp.where((e > cur) & (nt_ref[e] > 0), jnp.int32(e), nxt)
    k = t - st_ref[cur]
    ntc = nt_ref[cur]
    for i in range(3 * N_CHUNK):
        @pl.when(active & (nxt < 9) & (i * ntc >= 3 * N_CHUNK * k)
                 & (i * ntc < 3 * N_CHUNK * (k + 1)))
        def _(i=i):
            start_chunk(nxt, 1 - slot, i)

    @pl.when(is_first & active)
    def _():
        pltpu.make_async_copy(w1_hbm.at[cur], wb1.at[slot],
                              sems.at[slot, 0]).wait()
        pltpu.make_async_copy(w3_hbm.at[cur], wb3.at[slot],
                              sems.at[slot, 1]).wait()
        pltpu.make_async_copy(w2_hbm.at[cur], wb2.at[slot],
                              sems.at[slot, 2]).wait()

    @pl.when(active)
    def _():
        xb = xs_ref[...]
        p1 = lax.dot_general(xb, wb1[slot], (((1,), (1,)), ((), ())),
                             preferred_element_type=jnp.float32)
        p3 = lax.dot_general(xb, wb3[slot], (((1,), (1,)), ((), ())),
                             preferred_element_type=jnp.float32)
        hh = (p1 / (1.0 + jnp.exp(-p1))) * p3
        y = lax.dot_general(hh, wb2[slot], (((1,), (1,)), ((), ())),
                            preferred_element_type=jnp.float32)
        ys_ref[...] = y * rw_ref[...]


def _gmm(xs, roww, W1, W2, W3, tgrp, tact, nt, st):
    rw = roww.reshape(M_CAP, 1)
    grid_spec = pltpu.PrefetchScalarGridSpec(
        num_scalar_prefetch=4,
        grid=(N_TILES,),
        in_specs=[
            pl.BlockSpec((TM, DIM), lambda t, tg, ta, nt, st: (t, 0)),
            pl.BlockSpec((TM, 1), lambda t, tg, ta, nt, st: (t, 0)),
            pl.BlockSpec(memory_space=pl.ANY),
            pl.BlockSpec(memory_space=pl.ANY),
            pl.BlockSpec(memory_space=pl.ANY),
        ],
        out_specs=pl.BlockSpec((TM, DIM), lambda t, tg, ta, nt, st: (t, 0)),
        scratch_shapes=[
            pltpu.VMEM((2, HIDDEN, DIM), jnp.float32),
            pltpu.VMEM((2, HIDDEN, DIM), jnp.float32),
            pltpu.VMEM((2, DIM, HIDDEN), jnp.float32),
            pltpu.SemaphoreType.DMA((2, 3)),
        ],
    )
    return pl.pallas_call(
        _gmm_body,
        grid_spec=grid_spec,
        out_shape=jax.ShapeDtypeStruct((M_CAP, DIM), jnp.float32),
    )(tgrp, tact, nt, st, xs, rw, W1, W3, W2)


def _combine_body(ys_hbm, pa_hbm, pb_hbm, out_hbm, ia_v, ib_v, ba_v, bb_v,
                  sem, sem2):
    wid = lax.axis_index("s") * 2 + lax.axis_index("c")
    pltpu.sync_copy(pa_hbm.at[wid], ia_v)
    pltpu.sync_copy(pb_hbm.at[wid], ib_v)
    ca = pltpu.async_copy(ys_hbm.at[ia_v], ba_v, sem)
    cb = pltpu.async_copy(ys_hbm.at[ib_v], bb_v, sem2)
    ca.wait()
    cb.wait()

    def rowbody(r, _):
        for ci in range(DIM // 16):
            sl = pl.ds(ci * 16, 16)
            ba_v[r, sl] = ba_v[r, sl] + bb_v[r, sl]
        return 0

    lax.fori_loop(0, C_ROWS, rowbody, 0)
    pltpu.sync_copy(ba_v, out_hbm.at[pl.ds(wid * C_ROWS, C_ROWS)])


def _combine(ys, pa, pb):
    return pl.kernel(
        _combine_body,
        out_type=jax.ShapeDtypeStruct((N_TOK, DIM), jnp.float32),
        mesh=plsc.VectorSubcoreMesh(**_SC_MESH),
        scratch_types=[
            pltpu.VMEM((C_ROWS,), jnp.int32),
            pltpu.VMEM((C_ROWS,), jnp.int32),
            pltpu.VMEM((C_ROWS, DIM), jnp.float32),
            pltpu.VMEM((C_ROWS, DIM), jnp.float32),
            pltpu.SemaphoreType.DMA,
            pltpu.SemaphoreType.DMA,
        ],
    )(ys, pa, pb)


@jax.jit
def kernel(x, Wg, W1, W2, W3):
    b, s, d = x.shape
    x2d = x.reshape(b * s, d)
    ev, wv = _gate(x2d, Wg)
    roww, pos, tgrp, tact, nt, st, xs = _route_scatter(
        ev.reshape(-1), wv.reshape(NW, M_ASN // NW), x2d)
    pa = pos[:N_TOK].reshape(NW, C_ROWS)
    pb = pos[N_TOK:].reshape(NW, C_ROWS)
    ys = _gmm(xs, roww, W1, W2, W3, tgrp, tact, nt, st)
    out = _combine(ys, pa, pb)
    return out.reshape(b, s, d)
